# baseline (speedup 1.0000x reference)
import functools

import jax
import jax.numpy as jnp
from jax.experimental import pallas as pl
from jax.experimental.pallas import tpu as pltpu

_BATCH = 32
_L = 3
_TPS = _L + 1
_ROWS = _BATCH * _TPS
_HID = 1024
_VOCAB = 100000

_TILE_A = 8192
_TILE_B = 2048

_HIGH = jax.lax.Precision.HIGHEST
_BIG_I32 = 2**30


def _tile_argmax(x, col0, tile):
    col = col0 + jax.lax.broadcasted_iota(jnp.int32, x.shape, 1)
    x = jnp.where(col < _VOCAB, x, -jnp.inf)
    tmax = jnp.max(x, axis=1, keepdims=True)
    tidx = jnp.min(jnp.where(x == tmax, col, _BIG_I32), axis=1, keepdims=True)
    return tmax, tidx


def _phase_a_body(nva, dp_ref, logits_ref, gid_ref, nacc_ref, last_ref,
                  vmax_ref, vidx_ref):
    i = pl.program_id(0)

    @pl.when(i == 0)
    def _init():
        vmax_ref[:] = jnp.full((_ROWS, 1), -jnp.inf, jnp.float32)
        vidx_ref[:] = jnp.zeros((_ROWS, 1), jnp.int32)

    tmax, tidx = _tile_argmax(logits_ref[:], i * _TILE_A, _TILE_A)
    upd = tmax > vmax_ref[:]
    vidx_ref[:] = jnp.where(upd, tidx, vidx_ref[:])
    vmax_ref[:] = jnp.maximum(tmax, vmax_ref[:])

    @pl.when(i == nva - 1)
    def _finish():
        target = vidx_ref[:]
        m = (dp_ref[:] == target).astype(jnp.float32)
        r2 = jax.lax.broadcasted_iota(jnp.int32, (_ROWS, _ROWS), 0)
        c2 = jax.lax.broadcasted_iota(jnp.int32, (_ROWS, _ROWS), 1)
        tri = ((r2 // _TPS == c2 // _TPS) & (c2 <= r2)).astype(jnp.float32)
        miss = jnp.dot(tri, 1.0 - m, precision=_HIGH,
                       preferred_element_type=jnp.float32)
        prefix = (miss == 0.0).astype(jnp.float32)
        rb = jax.lax.broadcasted_iota(jnp.int32, (_BATCH, _ROWS), 0)
        cb = jax.lax.broadcasted_iota(jnp.int32, (_BATCH, _ROWS), 1)
        agg = ((cb // _TPS == rb) & (cb % _TPS < _L)).astype(jnp.float32)
        n_acc = 1 + jnp.dot(agg, prefix, precision=_HIGH,
                            preferred_element_type=jnp.float32).astype(jnp.int32)
        bidx = jax.lax.broadcasted_iota(jnp.int32, (_BATCH, 1), 0)
        gid = _TPS * bidx + n_acc - 1
        onehot = (cb == gid).astype(jnp.float32)
        last = jnp.dot(onehot, target.astype(jnp.float32), precision=_HIGH,
                       preferred_element_type=jnp.float32)
        gid_ref[:] = gid
        nacc_ref[:] = n_acc
        last_ref[:] = last.astype(jnp.int32)


def _phase_b_body(nvb, gid_ref, hs_ref, w_ref, lm_ref, tok_ref,
                  h_ref, vmax_ref, vidx_ref):
    i = pl.program_id(0)

    @pl.when(i == 0)
    def _init():
        cb = jax.lax.broadcasted_iota(jnp.int32, (_BATCH, _ROWS), 1)
        onehot = (cb == gid_ref[:]).astype(jnp.float32)
        h = jnp.dot(onehot, hs_ref[:], precision=_HIGH,
                    preferred_element_type=jnp.float32)
        hs = []
        for _ in range(_L):
            h = jnp.tanh(jnp.dot(h, w_ref[:], precision=_HIGH,
                                 preferred_element_type=jnp.float32))
            hs.append(h)
        h_ref[:] = jnp.concatenate(hs, axis=0)
        vmax_ref[:] = jnp.full((_L * _BATCH, 1), -jnp.inf, jnp.float32)
        vidx_ref[:] = jnp.zeros((_L * _BATCH, 1), jnp.int32)

    a = jnp.dot(h_ref[:], lm_ref[:], precision=_HIGH,
                preferred_element_type=jnp.float32)
    tmax, tidx = _tile_argmax(a, i * _TILE_B, _TILE_B)
    upd = tmax > vmax_ref[:]
    vidx_ref[:] = jnp.where(upd, tidx, vidx_ref[:])
    vmax_ref[:] = jnp.maximum(tmax, vmax_ref[:])

    @pl.when(i == nvb - 1)
    def _finish():
        tok_ref[:] = vidx_ref[:]


def kernel(logits, hidden_states, lm_head, W, draft_tokens):
    dp = jnp.concatenate(
        [draft_tokens, jnp.full((_BATCH, 1), -1, jnp.int32)], axis=1
    ).reshape(_ROWS, 1)

    nva = pl.cdiv(_VOCAB, _TILE_A)
    gid, n_acc, last = pl.pallas_call(
        functools.partial(_phase_a_body, nva),
        grid=(nva,),
        in_specs=[
            pl.BlockSpec((_ROWS, 1), lambda i: (0, 0)),
            pl.BlockSpec((_ROWS, _TILE_A), lambda i: (0, i)),
        ],
        out_specs=[
            pl.BlockSpec((_BATCH, 1), lambda i: (0, 0)),
            pl.BlockSpec((_BATCH, 1), lambda i: (0, 0)),
            pl.BlockSpec((_BATCH, 1), lambda i: (0, 0)),
        ],
        out_shape=[
            jax.ShapeDtypeStruct((_BATCH, 1), jnp.int32),
            jax.ShapeDtypeStruct((_BATCH, 1), jnp.int32),
            jax.ShapeDtypeStruct((_BATCH, 1), jnp.int32),
        ],
        scratch_shapes=[
            pltpu.VMEM((_ROWS, 1), jnp.float32),
            pltpu.VMEM((_ROWS, 1), jnp.int32),
        ],
        compiler_params=pltpu.CompilerParams(
            dimension_semantics=("arbitrary",),
        ),
    )(dp, logits)

    nvb = pl.cdiv(_VOCAB, _TILE_B)
    tok = pl.pallas_call(
        functools.partial(_phase_b_body, nvb),
        grid=(nvb,),
        in_specs=[
            pl.BlockSpec((_BATCH, 1), lambda i: (0, 0)),
            pl.BlockSpec((_ROWS, _HID), lambda i: (0, 0)),
            pl.BlockSpec((_HID, _HID), lambda i: (0, 0)),
            pl.BlockSpec((_HID, _TILE_B), lambda i: (0, i)),
        ],
        out_specs=pl.BlockSpec((_L * _BATCH, 1), lambda i: (0, 0)),
        out_shape=jax.ShapeDtypeStruct((_L * _BATCH, 1), jnp.int32),
        scratch_shapes=[
            pltpu.VMEM((_L * _BATCH, _HID), jnp.float32),
            pltpu.VMEM((_L * _BATCH, 1), jnp.float32),
            pltpu.VMEM((_L * _BATCH, 1), jnp.int32),
        ],
        compiler_params=pltpu.CompilerParams(
            dimension_semantics=("arbitrary",),
        ),
    )(gid, hidden_states, W, lm_head)

    stacked = tok.reshape(_L, _BATCH).T
    next_new = jnp.concatenate([last, stacked], axis=1)
    return next_new, stacked, n_acc.reshape(_BATCH)


# speedup vs baseline: 1.2191x; 1.2191x over previous
import functools

import jax
import jax.numpy as jnp
from jax.experimental import pallas as pl
from jax.experimental.pallas import tpu as pltpu

_BATCH = 32
_L = 3
_TPS = _L + 1
_ROWS = _BATCH * _TPS
_HID = 1024
_VOCAB = 100000

_TILE_A = 8192
_TILE_B = 2048

_HIGH = jax.lax.Precision.HIGHEST
_BIG_I32 = 2**30


def _tile_argmax(x, col0, tile):
    col = col0 + jax.lax.broadcasted_iota(jnp.int32, x.shape, 1)
    x = jnp.where(col < _VOCAB, x, -jnp.inf)
    tmax = jnp.max(x, axis=1, keepdims=True)
    tidx = jnp.min(jnp.where(x == tmax, col, _BIG_I32), axis=1, keepdims=True)
    return tmax, tidx


def _phase_a_body(nva, dp_ref, logits_ref, gid_ref, nacc_ref, last_ref,
                  vmax_ref, vidx_ref):
    i = pl.program_id(0)

    @pl.when(i == 0)
    def _init():
        vmax_ref[:] = jnp.full((_ROWS, 1), -jnp.inf, jnp.float32)
        vidx_ref[:] = jnp.zeros((_ROWS, 1), jnp.int32)

    tmax, tidx = _tile_argmax(logits_ref[:], i * _TILE_A, _TILE_A)
    upd = tmax > vmax_ref[:]
    vidx_ref[:] = jnp.where(upd, tidx, vidx_ref[:])
    vmax_ref[:] = jnp.maximum(tmax, vmax_ref[:])

    @pl.when(i == nva - 1)
    def _finish():
        target = vidx_ref[:]
        m = (dp_ref[:] == target).astype(jnp.float32)
        r2 = jax.lax.broadcasted_iota(jnp.int32, (_ROWS, _ROWS), 0)
        c2 = jax.lax.broadcasted_iota(jnp.int32, (_ROWS, _ROWS), 1)
        tri = ((r2 // _TPS == c2 // _TPS) & (c2 <= r2)).astype(jnp.float32)
        miss = jnp.dot(tri, 1.0 - m, precision=_HIGH,
                       preferred_element_type=jnp.float32)
        prefix = (miss == 0.0).astype(jnp.float32)
        rb = jax.lax.broadcasted_iota(jnp.int32, (_BATCH, _ROWS), 0)
        cb = jax.lax.broadcasted_iota(jnp.int32, (_BATCH, _ROWS), 1)
        agg = ((cb // _TPS == rb) & (cb % _TPS < _L)).astype(jnp.float32)
        n_acc = 1 + jnp.dot(agg, prefix, precision=_HIGH,
                            preferred_element_type=jnp.float32).astype(jnp.int32)
        bidx = jax.lax.broadcasted_iota(jnp.int32, (_BATCH, 1), 0)
        gid = _TPS * bidx + n_acc - 1
        onehot = (cb == gid).astype(jnp.float32)
        last = jnp.dot(onehot, target.astype(jnp.float32), precision=_HIGH,
                       preferred_element_type=jnp.float32)
        gid_ref[:] = gid
        nacc_ref[:] = n_acc
        last_ref[:] = last.astype(jnp.int32)


def _phase_b_body(nvb, gid_ref, hs_ref, w_ref, lm_ref, tok_ref,
                  h_ref, vmax_ref, vidx_ref):
    i = pl.program_id(0)

    @pl.when(i == 0)
    def _init():
        cb = jax.lax.broadcasted_iota(jnp.int32, (_BATCH, _ROWS), 1)
        onehot = (cb == gid_ref[:]).astype(jnp.float32)
        h = jnp.dot(onehot, hs_ref[:], precision=_HIGH,
                    preferred_element_type=jnp.float32)
        hs = []
        for _ in range(_L):
            h = jnp.tanh(jnp.dot(h, w_ref[:],
                                 preferred_element_type=jnp.float32))
            hs.append(h)
        h_ref[:] = jnp.concatenate(hs, axis=0)
        vmax_ref[:] = jnp.full((_L * _BATCH, 1), -jnp.inf, jnp.float32)
        vidx_ref[:] = jnp.zeros((_L * _BATCH, 1), jnp.int32)

    a = jnp.dot(h_ref[:], lm_ref[:],
                preferred_element_type=jnp.float32)
    tmax, tidx = _tile_argmax(a, i * _TILE_B, _TILE_B)
    upd = tmax > vmax_ref[:]
    vidx_ref[:] = jnp.where(upd, tidx, vidx_ref[:])
    vmax_ref[:] = jnp.maximum(tmax, vmax_ref[:])

    @pl.when(i == nvb - 1)
    def _finish():
        tok_ref[:] = vidx_ref[:]


def kernel(logits, hidden_states, lm_head, W, draft_tokens):
    dp = jnp.concatenate(
        [draft_tokens, jnp.full((_BATCH, 1), -1, jnp.int32)], axis=1
    ).reshape(_ROWS, 1)

    nva = pl.cdiv(_VOCAB, _TILE_A)
    gid, n_acc, last = pl.pallas_call(
        functools.partial(_phase_a_body, nva),
        grid=(nva,),
        in_specs=[
            pl.BlockSpec((_ROWS, 1), lambda i: (0, 0)),
            pl.BlockSpec((_ROWS, _TILE_A), lambda i: (0, i)),
        ],
        out_specs=[
            pl.BlockSpec((_BATCH, 1), lambda i: (0, 0)),
            pl.BlockSpec((_BATCH, 1), lambda i: (0, 0)),
            pl.BlockSpec((_BATCH, 1), lambda i: (0, 0)),
        ],
        out_shape=[
            jax.ShapeDtypeStruct((_BATCH, 1), jnp.int32),
            jax.ShapeDtypeStruct((_BATCH, 1), jnp.int32),
            jax.ShapeDtypeStruct((_BATCH, 1), jnp.int32),
        ],
        scratch_shapes=[
            pltpu.VMEM((_ROWS, 1), jnp.float32),
            pltpu.VMEM((_ROWS, 1), jnp.int32),
        ],
        compiler_params=pltpu.CompilerParams(
            dimension_semantics=("arbitrary",),
        ),
    )(dp, logits)

    nvb = pl.cdiv(_VOCAB, _TILE_B)
    tok = pl.pallas_call(
        functools.partial(_phase_b_body, nvb),
        grid=(nvb,),
        in_specs=[
            pl.BlockSpec((_BATCH, 1), lambda i: (0, 0)),
            pl.BlockSpec((_ROWS, _HID), lambda i: (0, 0)),
            pl.BlockSpec((_HID, _HID), lambda i: (0, 0)),
            pl.BlockSpec((_HID, _TILE_B), lambda i: (0, i)),
        ],
        out_specs=pl.BlockSpec((_L * _BATCH, 1), lambda i: (0, 0)),
        out_shape=jax.ShapeDtypeStruct((_L * _BATCH, 1), jnp.int32),
        scratch_shapes=[
            pltpu.VMEM((_L * _BATCH, _HID), jnp.float32),
            pltpu.VMEM((_L * _BATCH, 1), jnp.float32),
            pltpu.VMEM((_L * _BATCH, 1), jnp.int32),
        ],
        compiler_params=pltpu.CompilerParams(
            dimension_semantics=("arbitrary",),
        ),
    )(gid, hidden_states, W, lm_head)

    stacked = tok.reshape(_L, _BATCH).T
    next_new = jnp.concatenate([last, stacked], axis=1)
    return next_new, stacked, n_acc.reshape(_BATCH)
